# baseline (speedup 1.0000x reference)
import functools

import jax
import jax.numpy as jnp
from jax import lax
from jax.experimental import pallas as pl
from jax.experimental.pallas import tpu as pltpu
from jax.experimental.pallas import tpu_sc as plsc

_CUTOFF2 = 25.0
_K = 32
_NB = 16
_TM = 256
_CH = 64

_SC_CORES = 2
_SC_SUBCORES = 16
_NW = _SC_CORES * _SC_SUBCORES


def _topk_body(pos_l, bat_l, pos3, bat3, src_ref, wgt_ref,
               masked_ref, asrc_ref, awgt_ref):
    b = pl.program_id(0)
    n = pos_l.shape[1]
    nrb_tot = n // _TM
    inf = jnp.float32(jnp.inf)

    bat_full = bat_l[...]
    c0 = jnp.sum((bat_full < b).astype(jnp.int32))
    c1 = jnp.sum((bat_full <= b).astype(jnp.int32))
    qa0 = c0 // _CH
    qa1 = (c1 + _CH - 1) // _CH
    rb0 = c0 // _TM
    rb1 = (c1 + _TM - 1) // _TM

    iota_c = lax.broadcasted_iota(jnp.int32, (_CH, 1), 0)
    iota_l = lax.broadcasted_iota(jnp.int32, (1, _TM), 1)
    slot = lax.broadcasted_iota(jnp.int32, (_K, 1), 0)

    def row_block(rb, _):
        rs = rb * _TM
        xr = pos_l[0:1, pl.ds(rs, _TM)]
        yr = pos_l[1:2, pl.ds(rs, _TM)]
        zr = pos_l[2:3, pl.ds(rs, _TM)]
        brow = bat_l[0:1, pl.ds(rs, _TM)]
        row_ok = brow == b
        rowid = rs + iota_l

        def chunk_init(qa, _):
            xc = pos3[qa, :, 0:1]
            yc = pos3[qa, :, 1:2]
            zc = pos3[qa, :, 2:3]
            dx = xr - xc
            dy = yr - yc
            dz = zr - zc
            d2 = dx * dx + dy * dy + dz * dz
            valid = (bat3[qa] == b) & row_ok & (d2 <= _CUTOFF2)
            masked_ref[qa] = jnp.where(valid, d2, inf)
            return 0

        lax.fori_loop(qa0, qa1, chunk_init, 0)

        def pass_body(carry):
            t, _cont, vprev, jprev, src_acc, wgt_acc = carry

            def scan_chunk(qa, sc):
                m1, jm1, m2, jm2 = sc
                jj = iota_c + qa * _CH
                c = masked_ref[qa]
                elig = (c > vprev) | ((c == vprev) & (jj > jprev))
                ceff = jnp.where(elig, c, inf)
                v1 = jnp.min(ceff, axis=0, keepdims=True)
                jc1 = jnp.min(jnp.where(ceff == v1, jj, n),
                              axis=0, keepdims=True)
                ceff2 = jnp.where(jj == jc1, inf, ceff)
                v2 = jnp.min(ceff2, axis=0, keepdims=True)
                jc2 = jnp.min(jnp.where(ceff2 == v2, jj, n),
                              axis=0, keepdims=True)
                c1w = v1 < m1
                nm1 = jnp.where(c1w, v1, m1)
                nj1 = jnp.where(c1w, jc1, jm1)
                la = jnp.where(c1w, m1, v1)
                lj = jnp.where(c1w, jm1, jc1)
                lb = jnp.where(c1w, v2, m2)
                ljb = jnp.where(c1w, jc2, jm2)
                c2w = (lb < la) | ((lb == la) & (ljb < lj))
                nm2 = jnp.where(c2w, lb, la)
                nj2 = jnp.where(c2w, ljb, lj)
                return nm1, nj1, nm2, nj2

            big = jnp.full((1, _TM), inf, jnp.float32)
            bign = jnp.full((1, _TM), n, jnp.int32)
            v1, j1, v2, j2 = lax.fori_loop(
                qa0, qa1, scan_chunk, (big, bign, big, bign))

            def emit(t_slot, v, j, src_acc, wgt_acc):
                finite = v < inf
                srcv = jnp.where(finite, j, rowid)
                loop_m = srcv != rowid
                safe = jnp.sqrt(jnp.where(loop_m, v, 1.0))
                wv = jnp.where(loop_m, safe, 0.0)
                sel = slot == t_slot
                return (jnp.where(sel, srcv, src_acc),
                        jnp.where(sel, wv, wgt_acc))

            src_acc, wgt_acc = emit(t, v1, j1, src_acc, wgt_acc)
            src_acc, wgt_acc = emit(t + 1, v2, j2, src_acc, wgt_acc)
            cont = jnp.min(v2) < inf
            return t + 2, cont, v2, j2, src_acc, wgt_acc

        def cond(carry):
            return (carry[0] < _K) & carry[1]

        src0 = jnp.broadcast_to(rowid, (_K, _TM)).astype(jnp.int32)
        wgt0 = jnp.zeros((_K, _TM), jnp.float32)
        carry0 = (jnp.int32(0), jnp.bool_(True),
                  jnp.full((1, _TM), -jnp.inf, jnp.float32),
                  jnp.full((1, _TM), -1, jnp.int32), src0, wgt0)
        out = lax.while_loop(cond, pass_body, carry0)

        asrc_ref[rb] = jnp.where(row_ok, out[4], asrc_ref[rb])
        awgt_ref[rb] = jnp.where(row_ok, out[5], awgt_ref[rb])
        return 0

    lax.fori_loop(rb0, rb1, row_block, 0)

    @pl.when(b == _NB - 1)
    def _copy_out():
        def cp(rb, _):
            src_ref[rb] = asrc_ref[rb]
            wgt_ref[rb] = awgt_ref[rb]
            return 0
        lax.fori_loop(0, nrb_tot, cp, 0)


def _topk_call(pos, batch, interpret=False):
    n = pos.shape[0]
    pos_l = pos.T
    bat_l = batch.reshape(1, n)
    pos3 = pos.reshape(n // _CH, _CH, 3)
    bat3 = batch.reshape(n // _CH, _CH, 1)
    nrb = n // _TM
    srcT, wgtT = pl.pallas_call(
        _topk_body,
        grid=(_NB,),
        in_specs=[
            pl.BlockSpec((3, n), lambda i: (0, 0)),
            pl.BlockSpec((1, n), lambda i: (0, 0)),
            pl.BlockSpec((n // _CH, _CH, 3), lambda i: (0, 0, 0)),
            pl.BlockSpec((n // _CH, _CH, 1), lambda i: (0, 0, 0)),
        ],
        out_specs=[
            pl.BlockSpec((nrb, _K, _TM), lambda i: (0, 0, 0)),
            pl.BlockSpec((nrb, _K, _TM), lambda i: (0, 0, 0)),
        ],
        out_shape=[
            jax.ShapeDtypeStruct((nrb, _K, _TM), jnp.int32),
            jax.ShapeDtypeStruct((nrb, _K, _TM), jnp.float32),
        ],
        scratch_shapes=[
            pltpu.VMEM((n // _CH, _CH, _TM), jnp.float32),
            pltpu.VMEM((nrb, _K, _TM), jnp.int32),
            pltpu.VMEM((nrb, _K, _TM), jnp.float32),
        ],
        interpret=interpret,
    )(pos_l, bat_l, pos3, bat3)
    src2d = jnp.transpose(srcT, (0, 2, 1)).reshape(n, _K)
    wgt2d = jnp.transpose(wgtT, (0, 2, 1)).reshape(n, _K)
    return src2d, wgt2d


def _edge_vec_call(px, py, pz, src_flat):
    n = px.shape[0]
    e = src_flat.shape[0]
    epw = e // _NW
    mesh = plsc.VectorSubcoreMesh(
        core_axis_name="c", subcore_axis_name="s",
        num_cores=_SC_CORES, num_subcores=_SC_SUBCORES)

    @functools.partial(
        pl.kernel,
        mesh=mesh,
        compiler_params=pltpu.CompilerParams(needs_layout_passes=False),
        out_type=[jax.ShapeDtypeStruct((e,), jnp.float32)] * 3,
        scratch_types=[
            pltpu.VMEM((n,), jnp.float32),
            pltpu.VMEM((n,), jnp.float32),
            pltpu.VMEM((n,), jnp.float32),
            pltpu.VMEM((epw,), jnp.int32),
            pltpu.VMEM((epw,), jnp.float32),
            pltpu.VMEM((epw,), jnp.float32),
            pltpu.VMEM((epw,), jnp.float32),
        ],
    )
    def k(px_hbm, py_hbm, pz_hbm, src_hbm, vx_hbm, vy_hbm, vz_hbm,
          px_v, py_v, pz_v, src_v, vx_v, vy_v, vz_v):
        c = lax.axis_index("c")
        s = lax.axis_index("s")
        wid = s * _SC_CORES + c
        base = wid * epw
        pltpu.sync_copy(px_hbm, px_v)
        pltpu.sync_copy(py_hbm, py_v)
        pltpu.sync_copy(pz_hbm, pz_v)
        pltpu.sync_copy(src_hbm.at[pl.ds(base, epw)], src_v)
        lane = lax.iota(jnp.int32, 16)

        def body(t, _):
            off = t * 16
            j = src_v[pl.ds(off, 16)]
            i = lax.shift_right_logical(base + off + lane, 5)
            vx_v[pl.ds(off, 16)] = (plsc.load_gather(px_v, [j])
                                    - plsc.load_gather(px_v, [i]))
            vy_v[pl.ds(off, 16)] = (plsc.load_gather(py_v, [j])
                                    - plsc.load_gather(py_v, [i]))
            vz_v[pl.ds(off, 16)] = (plsc.load_gather(pz_v, [j])
                                    - plsc.load_gather(pz_v, [i]))
            return 0

        lax.fori_loop(0, epw // 16, body, 0)
        pltpu.sync_copy(vx_v, vx_hbm.at[pl.ds(base, epw)])
        pltpu.sync_copy(vy_v, vy_hbm.at[pl.ds(base, epw)])
        pltpu.sync_copy(vz_v, vz_hbm.at[pl.ds(base, epw)])

    return k(px, py, pz, src_flat)


def kernel(pos, batch):
    n = pos.shape[0]
    src2d, wgt2d = _topk_call(pos, batch)
    src_flat = src2d.reshape(-1)
    px = pos[:, 0]
    py = pos[:, 1]
    pz = pos[:, 2]
    vx, vy, vz = _edge_vec_call(px, py, pz, src_flat)
    edge_vec = jnp.stack([vx, vy, vz], axis=-1)
    dst = jnp.broadcast_to(
        jnp.arange(n, dtype=jnp.int32)[:, None], (n, _K)).reshape(-1)
    edge_index = jnp.stack([src_flat, dst], axis=0)
    return edge_index, wgt2d.reshape(-1), edge_vec


# speedup vs baseline: 1.1867x; 1.1867x over previous
import functools

import jax
import jax.numpy as jnp
from jax import lax
from jax.experimental import pallas as pl
from jax.experimental.pallas import tpu as pltpu
from jax.experimental.pallas import tpu_sc as plsc

_CUTOFF2 = 25.0
_K = 32
_NB = 16
_TM = 256
_CH = 128

_SC_CORES = 2
_SC_SUBCORES = 16
_NW = _SC_CORES * _SC_SUBCORES


def _topk_body(pos_l, bat_l, pos3, bat3, src_ref, wgt_ref,
               masked_ref, asrc_ref, awgt_ref):
    b = pl.program_id(0)
    n = pos_l.shape[1]
    nrb_tot = n // _TM
    inf = jnp.float32(jnp.inf)

    bat_full = bat_l[...]
    c0 = jnp.sum((bat_full < b).astype(jnp.int32))
    c1 = jnp.sum((bat_full <= b).astype(jnp.int32))
    qa0 = c0 // _CH
    qa1 = (c1 + _CH - 1) // _CH
    rb0 = c0 // _TM
    rb1 = (c1 + _TM - 1) // _TM

    iota_c = lax.broadcasted_iota(jnp.int32, (_CH, 1), 0)
    iota_l = lax.broadcasted_iota(jnp.int32, (1, _TM), 1)
    slot = lax.broadcasted_iota(jnp.int32, (_K, 1), 0)

    def row_block(rb, _):
        rs = rb * _TM
        xr = pos_l[0:1, pl.ds(rs, _TM)]
        yr = pos_l[1:2, pl.ds(rs, _TM)]
        zr = pos_l[2:3, pl.ds(rs, _TM)]
        brow = bat_l[0:1, pl.ds(rs, _TM)]
        row_ok = brow == b
        rowid = rs + iota_l

        def chunk_init(qa, _):
            xc = pos3[qa, :, 0:1]
            yc = pos3[qa, :, 1:2]
            zc = pos3[qa, :, 2:3]
            dx = xr - xc
            dy = yr - yc
            dz = zr - zc
            d2 = dx * dx + dy * dy + dz * dz
            valid = (bat3[qa] == b) & row_ok & (d2 <= _CUTOFF2)
            masked_ref[qa] = jnp.where(valid, d2, inf)
            return 0

        lax.fori_loop(qa0, qa1, chunk_init, 0)

        def pass_body(carry):
            t, _cont, vprev, jprev, src_acc, wgt_acc = carry

            def scan_chunk(qa, sc):
                m1v, m1j, m2v, m2j, m3v, m3j = sc
                jj = iota_c + qa * _CH
                c = masked_ref[qa]
                elig = (c > vprev) | ((c == vprev) & (jj > jprev))
                ceff = jnp.where(elig, c, inf)
                v1 = jnp.min(ceff, axis=0, keepdims=True)
                jc1 = jnp.min(jnp.where(ceff == v1, jj, n),
                              axis=0, keepdims=True)
                ceff2 = jnp.where(jj == jc1, inf, ceff)
                v2 = jnp.min(ceff2, axis=0, keepdims=True)
                jc2 = jnp.min(jnp.where(ceff2 == v2, jj, n),
                              axis=0, keepdims=True)
                ceff3 = jnp.where(jj == jc2, inf, ceff2)
                v3 = jnp.min(ceff3, axis=0, keepdims=True)
                jc3 = jnp.min(jnp.where(ceff3 == v3, jj, n),
                              axis=0, keepdims=True)
                for xv, xj in ((v1, jc1), (v2, jc2), (v3, jc3)):
                    b1 = (xv < m1v) | ((xv == m1v) & (xj < m1j))
                    b2 = (xv < m2v) | ((xv == m2v) & (xj < m2j))
                    b3 = (xv < m3v) | ((xv == m3v) & (xj < m3j))
                    nm1v = jnp.where(b1, xv, m1v)
                    nm1j = jnp.where(b1, xj, m1j)
                    nm2v = jnp.where(b1, m1v, jnp.where(b2, xv, m2v))
                    nm2j = jnp.where(b1, m1j, jnp.where(b2, xj, m2j))
                    nm3v = jnp.where(b2, m2v, jnp.where(b3, xv, m3v))
                    nm3j = jnp.where(b2, m2j, jnp.where(b3, xj, m3j))
                    m1v, m1j, m2v, m2j, m3v, m3j = (
                        nm1v, nm1j, nm2v, nm2j, nm3v, nm3j)
                return m1v, m1j, m2v, m2j, m3v, m3j

            big = jnp.full((1, _TM), inf, jnp.float32)
            bign = jnp.full((1, _TM), n, jnp.int32)
            v1, j1, v2, j2, v3, j3 = lax.fori_loop(
                qa0, qa1, scan_chunk, (big, bign, big, bign, big, bign))

            def emit(t_slot, v, j, src_acc, wgt_acc):
                finite = v < inf
                srcv = jnp.where(finite, j, rowid)
                loop_m = srcv != rowid
                safe = jnp.sqrt(jnp.where(loop_m, v, 1.0))
                wv = jnp.where(loop_m, safe, 0.0)
                sel = slot == t_slot
                return (jnp.where(sel, srcv, src_acc),
                        jnp.where(sel, wv, wgt_acc))

            src_acc, wgt_acc = emit(t, v1, j1, src_acc, wgt_acc)
            src_acc, wgt_acc = emit(t + 1, v2, j2, src_acc, wgt_acc)
            src_acc, wgt_acc = emit(t + 2, v3, j3, src_acc, wgt_acc)
            cont = jnp.min(v3) < inf
            return t + 3, cont, v3, j3, src_acc, wgt_acc

        def cond(carry):
            return (carry[0] < _K) & carry[1]

        src0 = jnp.broadcast_to(rowid, (_K, _TM)).astype(jnp.int32)
        wgt0 = jnp.zeros((_K, _TM), jnp.float32)
        carry0 = (jnp.int32(0), jnp.bool_(True),
                  jnp.full((1, _TM), -jnp.inf, jnp.float32),
                  jnp.full((1, _TM), -1, jnp.int32), src0, wgt0)
        out = lax.while_loop(cond, pass_body, carry0)

        asrc_ref[rb] = jnp.where(row_ok, out[4], asrc_ref[rb])
        awgt_ref[rb] = jnp.where(row_ok, out[5], awgt_ref[rb])
        return 0

    lax.fori_loop(rb0, rb1, row_block, 0)

    @pl.when(b == _NB - 1)
    def _copy_out():
        def cp(rb, _):
            src_ref[rb] = asrc_ref[rb]
            wgt_ref[rb] = awgt_ref[rb]
            return 0
        lax.fori_loop(0, nrb_tot, cp, 0)


def _topk_call(pos, batch, interpret=False):
    n = pos.shape[0]
    pos_l = pos.T
    bat_l = batch.reshape(1, n)
    pos3 = pos.reshape(n // _CH, _CH, 3)
    bat3 = batch.reshape(n // _CH, _CH, 1)
    nrb = n // _TM
    srcT, wgtT = pl.pallas_call(
        _topk_body,
        grid=(_NB,),
        in_specs=[
            pl.BlockSpec((3, n), lambda i: (0, 0)),
            pl.BlockSpec((1, n), lambda i: (0, 0)),
            pl.BlockSpec((n // _CH, _CH, 3), lambda i: (0, 0, 0)),
            pl.BlockSpec((n // _CH, _CH, 1), lambda i: (0, 0, 0)),
        ],
        out_specs=[
            pl.BlockSpec((nrb, _K, _TM), lambda i: (0, 0, 0)),
            pl.BlockSpec((nrb, _K, _TM), lambda i: (0, 0, 0)),
        ],
        out_shape=[
            jax.ShapeDtypeStruct((nrb, _K, _TM), jnp.int32),
            jax.ShapeDtypeStruct((nrb, _K, _TM), jnp.float32),
        ],
        scratch_shapes=[
            pltpu.VMEM((n // _CH, _CH, _TM), jnp.float32),
            pltpu.VMEM((nrb, _K, _TM), jnp.int32),
            pltpu.VMEM((nrb, _K, _TM), jnp.float32),
        ],
        interpret=interpret,
    )(pos_l, bat_l, pos3, bat3)
    src2d = jnp.transpose(srcT, (0, 2, 1)).reshape(n, _K)
    wgt2d = jnp.transpose(wgtT, (0, 2, 1)).reshape(n, _K)
    return src2d, wgt2d


def _edge_vec_call(px, py, pz, src_flat):
    n = px.shape[0]
    e = src_flat.shape[0]
    epw = e // _NW
    mesh = plsc.VectorSubcoreMesh(
        core_axis_name="c", subcore_axis_name="s",
        num_cores=_SC_CORES, num_subcores=_SC_SUBCORES)

    @functools.partial(
        pl.kernel,
        mesh=mesh,
        compiler_params=pltpu.CompilerParams(needs_layout_passes=False),
        out_type=[jax.ShapeDtypeStruct((e,), jnp.float32)] * 3,
        scratch_types=[
            pltpu.VMEM((n,), jnp.float32),
            pltpu.VMEM((n,), jnp.float32),
            pltpu.VMEM((n,), jnp.float32),
            pltpu.VMEM((epw,), jnp.int32),
            pltpu.VMEM((epw,), jnp.float32),
            pltpu.VMEM((epw,), jnp.float32),
            pltpu.VMEM((epw,), jnp.float32),
        ],
    )
    def k(px_hbm, py_hbm, pz_hbm, src_hbm, vx_hbm, vy_hbm, vz_hbm,
          px_v, py_v, pz_v, src_v, vx_v, vy_v, vz_v):
        c = lax.axis_index("c")
        s = lax.axis_index("s")
        wid = s * _SC_CORES + c
        base = wid * epw
        pltpu.sync_copy(px_hbm, px_v)
        pltpu.sync_copy(py_hbm, py_v)
        pltpu.sync_copy(pz_hbm, pz_v)
        pltpu.sync_copy(src_hbm.at[pl.ds(base, epw)], src_v)
        lane = lax.iota(jnp.int32, 16)

        def body(t, _):
            off = t * 16
            j = src_v[pl.ds(off, 16)]
            i = lax.shift_right_logical(base + off + lane, 5)
            vx_v[pl.ds(off, 16)] = (plsc.load_gather(px_v, [j])
                                    - plsc.load_gather(px_v, [i]))
            vy_v[pl.ds(off, 16)] = (plsc.load_gather(py_v, [j])
                                    - plsc.load_gather(py_v, [i]))
            vz_v[pl.ds(off, 16)] = (plsc.load_gather(pz_v, [j])
                                    - plsc.load_gather(pz_v, [i]))
            return 0

        lax.fori_loop(0, epw // 16, body, 0)
        pltpu.sync_copy(vx_v, vx_hbm.at[pl.ds(base, epw)])
        pltpu.sync_copy(vy_v, vy_hbm.at[pl.ds(base, epw)])
        pltpu.sync_copy(vz_v, vz_hbm.at[pl.ds(base, epw)])

    return k(px, py, pz, src_flat)


def kernel(pos, batch):
    n = pos.shape[0]
    src2d, wgt2d = _topk_call(pos, batch)
    src_flat = src2d.reshape(-1)
    px = pos[:, 0]
    py = pos[:, 1]
    pz = pos[:, 2]
    vx, vy, vz = _edge_vec_call(px, py, pz, src_flat)
    edge_vec = jnp.stack([vx, vy, vz], axis=-1)
    dst = jnp.broadcast_to(
        jnp.arange(n, dtype=jnp.int32)[:, None], (n, _K)).reshape(-1)
    edge_index = jnp.stack([src_flat, dst], axis=0)
    return edge_index, wgt2d.reshape(-1), edge_vec


# speedup vs baseline: 1.1973x; 1.0089x over previous
import functools

import jax
import jax.numpy as jnp
from jax import lax
from jax.experimental import pallas as pl
from jax.experimental.pallas import tpu as pltpu
from jax.experimental.pallas import tpu_sc as plsc

_CUTOFF2 = 25.0
_K = 32
_NB = 16
_TM = 256
_CH = 128
_EPP = 4

_SC_CORES = 2
_SC_SUBCORES = 16
_NW = _SC_CORES * _SC_SUBCORES


def _topk_body(pos_l, bat_l, pos3, bat3, src_ref, wgt_ref,
               masked_ref, asrc_ref, awgt_ref):
    b = pl.program_id(0)
    n = pos_l.shape[1]
    nrb_tot = n // _TM
    inf = jnp.float32(jnp.inf)

    bat_full = bat_l[...]
    c0 = jnp.sum((bat_full < b).astype(jnp.int32))
    c1 = jnp.sum((bat_full <= b).astype(jnp.int32))
    qa0 = c0 // _CH
    qa1 = (c1 + _CH - 1) // _CH
    rb0 = c0 // _TM
    rb1 = (c1 + _TM - 1) // _TM

    iota_c = lax.broadcasted_iota(jnp.int32, (_CH, 1), 0)
    iota_l = lax.broadcasted_iota(jnp.int32, (1, _TM), 1)
    slot = lax.broadcasted_iota(jnp.int32, (_K, 1), 0)

    def row_block(rb, _):
        rs = rb * _TM
        xr = pos_l[0:1, pl.ds(rs, _TM)]
        yr = pos_l[1:2, pl.ds(rs, _TM)]
        zr = pos_l[2:3, pl.ds(rs, _TM)]
        brow = bat_l[0:1, pl.ds(rs, _TM)]
        row_ok = brow == b
        rowid = rs + iota_l

        def chunk_init(qa, _):
            xc = pos3[qa, :, 0:1]
            yc = pos3[qa, :, 1:2]
            zc = pos3[qa, :, 2:3]
            dx = xr - xc
            dy = yr - yc
            dz = zr - zc
            d2 = dx * dx + dy * dy + dz * dz
            valid = (bat3[qa] == b) & row_ok & (d2 <= _CUTOFF2)
            masked_ref[qa] = jnp.where(valid, d2, inf)
            return 0

        lax.fori_loop(qa0, qa1, chunk_init, 0)

        def pass_body(carry):
            t, _cont, vprev, jprev, src_acc, wgt_acc = carry

            def scan_chunk(qa, sc):
                mv = list(sc[0::2])
                mj = list(sc[1::2])
                jj = iota_c + qa * _CH
                c = masked_ref[qa]
                elig = (c > vprev) | ((c == vprev) & (jj > jprev))
                ceff = jnp.where(elig, c, inf)
                cands = []
                for _ in range(_EPP):
                    v = jnp.min(ceff, axis=0, keepdims=True)
                    jc = jnp.min(jnp.where(ceff == v, jj, n),
                                 axis=0, keepdims=True)
                    cands.append((v, jc))
                    ceff = jnp.where(jj == jc, inf, ceff)
                for xv, xj in cands:
                    bs = [(xv < mv[k]) | ((xv == mv[k]) & (xj < mj[k]))
                          for k in range(_EPP)]
                    nv = [jnp.where(bs[0], xv, mv[0])]
                    nj = [jnp.where(bs[0], xj, mj[0])]
                    for k in range(1, _EPP):
                        nv.append(jnp.where(bs[k - 1], mv[k - 1],
                                            jnp.where(bs[k], xv, mv[k])))
                        nj.append(jnp.where(bs[k - 1], mj[k - 1],
                                            jnp.where(bs[k], xj, mj[k])))
                    mv, mj = nv, nj
                out = []
                for k in range(_EPP):
                    out.extend((mv[k], mj[k]))
                return tuple(out)

            big = jnp.full((1, _TM), inf, jnp.float32)
            bign = jnp.full((1, _TM), n, jnp.int32)
            res = lax.fori_loop(
                qa0, qa1, scan_chunk, (big, bign) * _EPP)

            def emit(t_slot, v, j, src_acc, wgt_acc):
                finite = v < inf
                srcv = jnp.where(finite, j, rowid)
                loop_m = srcv != rowid
                safe = jnp.sqrt(jnp.where(loop_m, v, 1.0))
                wv = jnp.where(loop_m, safe, 0.0)
                sel = slot == t_slot
                return (jnp.where(sel, srcv, src_acc),
                        jnp.where(sel, wv, wgt_acc))

            for k in range(_EPP):
                src_acc, wgt_acc = emit(t + k, res[2 * k], res[2 * k + 1],
                                        src_acc, wgt_acc)
            cont = jnp.min(res[-2]) < inf
            return t + _EPP, cont, res[-2], res[-1], src_acc, wgt_acc

        def cond(carry):
            return (carry[0] < _K) & carry[1]

        src0 = jnp.broadcast_to(rowid, (_K, _TM)).astype(jnp.int32)
        wgt0 = jnp.zeros((_K, _TM), jnp.float32)
        carry0 = (jnp.int32(0), jnp.bool_(True),
                  jnp.full((1, _TM), -jnp.inf, jnp.float32),
                  jnp.full((1, _TM), -1, jnp.int32), src0, wgt0)
        out = lax.while_loop(cond, pass_body, carry0)

        asrc_ref[rb] = jnp.where(row_ok, out[4], asrc_ref[rb])
        awgt_ref[rb] = jnp.where(row_ok, out[5], awgt_ref[rb])
        return 0

    lax.fori_loop(rb0, rb1, row_block, 0)

    @pl.when(b == _NB - 1)
    def _copy_out():
        def cp(rb, _):
            src_ref[rb] = asrc_ref[rb]
            wgt_ref[rb] = awgt_ref[rb]
            return 0
        lax.fori_loop(0, nrb_tot, cp, 0)


def _topk_call(pos, batch, interpret=False):
    n = pos.shape[0]
    pos_l = pos.T
    bat_l = batch.reshape(1, n)
    pos3 = pos.reshape(n // _CH, _CH, 3)
    bat3 = batch.reshape(n // _CH, _CH, 1)
    nrb = n // _TM
    srcT, wgtT = pl.pallas_call(
        _topk_body,
        grid=(_NB,),
        in_specs=[
            pl.BlockSpec((3, n), lambda i: (0, 0)),
            pl.BlockSpec((1, n), lambda i: (0, 0)),
            pl.BlockSpec((n // _CH, _CH, 3), lambda i: (0, 0, 0)),
            pl.BlockSpec((n // _CH, _CH, 1), lambda i: (0, 0, 0)),
        ],
        out_specs=[
            pl.BlockSpec((nrb, _K, _TM), lambda i: (0, 0, 0)),
            pl.BlockSpec((nrb, _K, _TM), lambda i: (0, 0, 0)),
        ],
        out_shape=[
            jax.ShapeDtypeStruct((nrb, _K, _TM), jnp.int32),
            jax.ShapeDtypeStruct((nrb, _K, _TM), jnp.float32),
        ],
        scratch_shapes=[
            pltpu.VMEM((n // _CH, _CH, _TM), jnp.float32),
            pltpu.VMEM((nrb, _K, _TM), jnp.int32),
            pltpu.VMEM((nrb, _K, _TM), jnp.float32),
        ],
        interpret=interpret,
    )(pos_l, bat_l, pos3, bat3)
    src2d = jnp.transpose(srcT, (0, 2, 1)).reshape(n, _K)
    wgt2d = jnp.transpose(wgtT, (0, 2, 1)).reshape(n, _K)
    return src2d, wgt2d


def _edge_vec_call(px, py, pz, src_flat):
    n = px.shape[0]
    e = src_flat.shape[0]
    epw = e // _NW
    mesh = plsc.VectorSubcoreMesh(
        core_axis_name="c", subcore_axis_name="s",
        num_cores=_SC_CORES, num_subcores=_SC_SUBCORES)

    @functools.partial(
        pl.kernel,
        mesh=mesh,
        compiler_params=pltpu.CompilerParams(needs_layout_passes=False),
        out_type=[jax.ShapeDtypeStruct((e,), jnp.float32)] * 3,
        scratch_types=[
            pltpu.VMEM((n,), jnp.float32),
            pltpu.VMEM((n,), jnp.float32),
            pltpu.VMEM((n,), jnp.float32),
            pltpu.VMEM((epw,), jnp.int32),
            pltpu.VMEM((epw,), jnp.float32),
            pltpu.VMEM((epw,), jnp.float32),
            pltpu.VMEM((epw,), jnp.float32),
        ],
    )
    def k(px_hbm, py_hbm, pz_hbm, src_hbm, vx_hbm, vy_hbm, vz_hbm,
          px_v, py_v, pz_v, src_v, vx_v, vy_v, vz_v):
        c = lax.axis_index("c")
        s = lax.axis_index("s")
        wid = s * _SC_CORES + c
        base = wid * epw
        pltpu.sync_copy(px_hbm, px_v)
        pltpu.sync_copy(py_hbm, py_v)
        pltpu.sync_copy(pz_hbm, pz_v)
        pltpu.sync_copy(src_hbm.at[pl.ds(base, epw)], src_v)
        lane = lax.iota(jnp.int32, 16)

        def body(t, _):
            off = t * 16
            j = src_v[pl.ds(off, 16)]
            i = lax.shift_right_logical(base + off + lane, 5)
            vx_v[pl.ds(off, 16)] = (plsc.load_gather(px_v, [j])
                                    - plsc.load_gather(px_v, [i]))
            vy_v[pl.ds(off, 16)] = (plsc.load_gather(py_v, [j])
                                    - plsc.load_gather(py_v, [i]))
            vz_v[pl.ds(off, 16)] = (plsc.load_gather(pz_v, [j])
                                    - plsc.load_gather(pz_v, [i]))
            return 0

        lax.fori_loop(0, epw // 16, body, 0)
        pltpu.sync_copy(vx_v, vx_hbm.at[pl.ds(base, epw)])
        pltpu.sync_copy(vy_v, vy_hbm.at[pl.ds(base, epw)])
        pltpu.sync_copy(vz_v, vz_hbm.at[pl.ds(base, epw)])

    return k(px, py, pz, src_flat)


def kernel(pos, batch):
    n = pos.shape[0]
    src2d, wgt2d = _topk_call(pos, batch)
    src_flat = src2d.reshape(-1)
    px = pos[:, 0]
    py = pos[:, 1]
    pz = pos[:, 2]
    vx, vy, vz = _edge_vec_call(px, py, pz, src_flat)
    edge_vec = jnp.stack([vx, vy, vz], axis=-1)
    dst = jnp.broadcast_to(
        jnp.arange(n, dtype=jnp.int32)[:, None], (n, _K)).reshape(-1)
    edge_index = jnp.stack([src_flat, dst], axis=0)
    return edge_index, wgt2d.reshape(-1), edge_vec
